# baseline (speedup 1.0000x reference)
import jax
import jax.numpy as jnp
from jax.experimental import pallas as pl


def _conv1d(x, w, b):
    y = jax.lax.conv_general_dilated(x, w, window_strides=(1,), padding='VALID',
                                     dimension_numbers=('NCH', 'OIH', 'NCH'))
    return y + b[None, :, None]


def _avgpool3(x):
    y = jax.lax.reduce_window(x, 0.0, jax.lax.add, (1, 1, 3), (1, 1, 3), 'VALID')
    return y / 3.0


def _seq_filter(p, idx):
    x = jnp.take(p['emb'], idx, axis=0)
    x = jnp.transpose(x, (0, 2, 1))
    x = jax.nn.relu(_avgpool3(_conv1d(x, p['c1w'], p['c1b'])))
    x = jax.nn.relu(_avgpool3(_conv1d(x, p['c2w'], p['c2b'])))
    outs = []
    for fw, fb in zip(p['fw'], p['fb']):
        outs.append(jnp.max(_conv1d(x, fw, fb), axis=2))
    x = jnp.concatenate(outs, axis=-1)
    x = jax.nn.relu(x @ p['l1w'].T + p['l1b'])
    return x @ p['l2w'].T + p['l2b']


def _gat_conv(x, src, dst, lp, n):
    h = x @ lp['gat_w'].T
    a = jax.nn.leaky_relu(jnp.take(h @ lp['att_src'], src) + jnp.take(h @ lp['att_dst'], dst),
                          negative_slope=0.2)
    m = jax.lax.stop_gradient(jax.ops.segment_max(a, dst, num_segments=n))
    e = jnp.exp(a - jnp.take(m, dst))
    denom = jax.ops.segment_sum(e, dst, num_segments=n)
    alpha = e / (jnp.take(denom, dst) + 1e-16)
    out = jax.ops.segment_sum(jnp.take(h, src, axis=0) * alpha[:, None], dst, num_segments=n)
    return out + lp['gat_b']


def _sage_conv(x, src, dst, lp, n):
    s = jax.ops.segment_sum(jnp.take(x, src, axis=0), dst, num_segments=n)
    cnt = jax.ops.segment_sum(jnp.ones(src.shape, dtype=x.dtype), dst, num_segments=n)
    mean = s / jnp.maximum(cnt, 1.0)[:, None]
    return mean @ lp['sage_lw'].T + lp['sage_lb'] + x @ lp['sage_rw'].T


def _identity_pallas(x):
    def body(x_ref, o_ref):
        o_ref[...] = x_ref[...]
    return pl.pallas_call(body, out_shape=jax.ShapeDtypeStruct(x.shape, x.dtype))(x)


def kernel(params, drug_features, target_features, edges, all_edges):
    df = _seq_filter(params['drug'], drug_features)
    tf = _seq_filter(params['target'], target_features)
    x = jnp.concatenate([df, tf], axis=0)
    n = x.shape[0]
    loop = jnp.arange(n, dtype=all_edges.dtype)
    src_sl = jnp.concatenate([all_edges[0], loop])
    dst_sl = jnp.concatenate([all_edges[1], loop])
    for lp in params['layers']:
        g = _gat_conv(x, src_sl, dst_sl, lp, n)
        s = _sage_conv(x, all_edges[0], all_edges[1], lp, n)
        x = jnp.tanh(jnp.concatenate([x, g, s], axis=-1) @ lp['lin_w'].T + lp['lin_b'])
    logits = jnp.sum(jnp.take(x, edges[0], axis=0) * jnp.take(x, edges[1], axis=0), axis=-1)
    return jax.nn.sigmoid(_identity_pallas(logits))


# speedup vs baseline: 1.6784x; 1.6784x over previous
import functools
import jax
import jax.numpy as jnp
from jax import lax
from jax.experimental import pallas as pl
from jax.experimental.pallas import tpu as pltpu
from jax.experimental.pallas import tpu_sc as plsc

N = 10000
NPAD = 10240
D = 128
NT = 16
DCHUNK = 128

E_DEC = 100000
DEC_CH_PW = 25
E_DEC_PAD = DEC_CH_PW * DCHUNK * 32

_mesh = plsc.VectorSubcoreMesh(core_axis_name="c", subcore_axis_name="s")


@functools.partial(
    pl.kernel,
    mesh=_mesh,
    compiler_params=pltpu.CompilerParams(needs_layout_passes=False),
    out_type=jax.ShapeDtypeStruct((E_DEC_PAD,), jnp.float32),
    scratch_types=[
        pltpu.VMEM((DCHUNK,), jnp.int32),
        pltpu.VMEM((DCHUNK,), jnp.int32),
        pltpu.VMEM((DCHUNK, D), jnp.float32),
        pltpu.VMEM((DCHUNK, D), jnp.float32),
        pltpu.VMEM((DCHUNK,), jnp.float32),
        pltpu.SemaphoreType.DMA,
    ],
)
def _decode_sc(x_hbm, e0_hbm, e1_hbm, out_hbm, av, bv, xa, xb, lbuf, sem):
    cid = lax.axis_index("c")
    sid = lax.axis_index("s")
    wid = cid * NT + sid
    wbase = wid * (DEC_CH_PW * DCHUNK)

    def chunk_body(j, carry):
        off = pl.multiple_of(wbase + j * DCHUNK, DCHUNK)
        pltpu.sync_copy(e0_hbm.at[pl.ds(off, DCHUNK)], av)
        pltpu.sync_copy(e1_hbm.at[pl.ds(off, DCHUNK)], bv)
        pltpu.async_copy(x_hbm.at[av], xa, sem).wait()
        pltpu.async_copy(x_hbm.at[bv], xb, sem).wait()

        lane = lax.iota(jnp.int32, 16)

        def rowdot(k, c):
            z = jnp.zeros((16,), jnp.float32)
            for l in range(16):
                i = k * 16 + l
                acc = xa[i, pl.ds(0, 16)] * xb[i, pl.ds(0, 16)]
                for q in range(1, 8):
                    acc = acc + xa[i, pl.ds(q * 16, 16)] * xb[i, pl.ds(q * 16, 16)]
                z = jnp.where(lane == l, jnp.sum(acc), z)
            lbuf[pl.ds(k * 16, 16)] = 1.0 / (1.0 + jnp.exp(-z))
            return c
        lax.fori_loop(0, DCHUNK // 16, rowdot, 0)

        pltpu.sync_copy(lbuf, out_hbm.at[pl.ds(off, DCHUNK)])
        return carry
    lax.fori_loop(0, DEC_CH_PW, chunk_body, 0)


def _pre_body(x_ref, wh_ref, vp_ref, h_ref, a_ref):
    x = x_ref[...]
    h_ref[...] = jnp.dot(x, wh_ref[...], preferred_element_type=jnp.float32)
    a_ref[...] = jnp.dot(x, vp_ref[...], preferred_element_type=jnp.float32)


_BLK = 256
_pre_tc = pl.pallas_call(
    _pre_body,
    grid=(NPAD // _BLK,),
    in_specs=[
        pl.BlockSpec((_BLK, D), lambda i: (i, 0)),
        pl.BlockSpec((D, D), lambda i: (0, 0)),
        pl.BlockSpec((D, D), lambda i: (0, 0)),
    ],
    out_specs=[pl.BlockSpec((_BLK, D), lambda i: (i, 0))] * 2,
    out_shape=[jax.ShapeDtypeStruct((NPAD, D), jnp.float32)] * 2,
)


def _post_body(x_ref, h_ref, accg_ref, accx_ref, a_ref, den_ref, cnt_ref,
               lw_ref, rw_ref, w1_ref, w2_ref, w3_ref,
               gatb_ref, sagb_ref, linb_ref, o_ref):
    x = x_ref[...]
    h = h_ref[...]
    asd = a_ref[:, 0:1] + a_ref[:, 1:2]
    es = jnp.exp(jnp.where(asd > 0, asd, asd * 0.2))
    den = den_ref[:, 0:1]
    g = (accg_ref[...] + es * h) / (den + es + 1e-16) + gatb_ref[...]
    cnt = jnp.maximum(cnt_ref[:, 0:1], 1.0)
    mean = accx_ref[...] / cnt
    s = (jnp.dot(mean, lw_ref[...], preferred_element_type=jnp.float32)
         + sagb_ref[...]
         + jnp.dot(x, rw_ref[...], preferred_element_type=jnp.float32))
    o = (jnp.dot(x, w1_ref[...], preferred_element_type=jnp.float32)
         + jnp.dot(g, w2_ref[...], preferred_element_type=jnp.float32)
         + jnp.dot(s, w3_ref[...], preferred_element_type=jnp.float32)
         + linb_ref[...])
    o_ref[...] = jnp.tanh(o)


_post_tc = pl.pallas_call(
    _post_body,
    grid=(NPAD // _BLK,),
    in_specs=(
        [pl.BlockSpec((_BLK, D), lambda i: (i, 0))] * 4
        + [pl.BlockSpec((_BLK, D), lambda i: (i, 0)),
           pl.BlockSpec((_BLK, 16), lambda i: (i, 0)),
           pl.BlockSpec((_BLK, 16), lambda i: (i, 0))]
        + [pl.BlockSpec((D, D), lambda i: (0, 0))] * 5
        + [pl.BlockSpec((1, D), lambda i: (0, 0))] * 3
    ),
    out_specs=pl.BlockSpec((_BLK, D), lambda i: (i, 0)),
    out_shape=jax.ShapeDtypeStruct((NPAD, D), jnp.float32),
)


NSEQ = 5000
EB = 40


def _make_encoder(T, V, fsizes):
    Vp = max(16, 1 << (V - 1).bit_length())
    L1 = T - 4
    P1 = L1 // 3
    L2 = P1 - 2
    P2 = L2 // 3
    OUT = 128

    def body(idx_ref, emb_ref, w1_ref, c1b_ref, w2_ref, c2b_ref,
             f0_ref, f1_ref, f2_ref, fb0_ref, fb1_ref, fb2_ref,
             l1w_ref, l1b_ref, l2w_ref, l2b_ref, o_ref):
        idx = idx_ref[...]
        oh = (idx == lax.broadcasted_iota(jnp.int32, (EB * T, Vp), 1)
              ).astype(jnp.float32)
        x = jnp.dot(oh, emb_ref[...], preferred_element_type=jnp.float32)
        y = None
        for j in range(5):
            z = jnp.dot(x, w1_ref[64 * j:64 * (j + 1), :],
                        preferred_element_type=jnp.float32)
            z = z.reshape(EB, T, 32)[:, j:j + L1, :]
            y = z if y is None else y + z
        y = y + c1b_ref[...][None]
        y = jax.nn.relu(y[:, :3 * P1, :].reshape(EB, P1, 3, 32).mean(axis=2))
        x2 = y.reshape(EB * P1, 32)
        y2 = None
        for j in range(3):
            z = jnp.dot(x2, w2_ref[32 * j:32 * (j + 1), :],
                        preferred_element_type=jnp.float32)
            z = z.reshape(EB, P1, 16)[:, j:j + L2, :]
            y2 = z if y2 is None else y2 + z
        y2 = y2 + c2b_ref[...][None]
        y2 = jax.nn.relu(y2[:, :3 * P2, :].reshape(EB, P2, 3, 16).mean(axis=2))
        x3 = y2.reshape(EB * P2, 16)
        outs = []
        for fr, fb, k in ((f0_ref, fb0_ref, fsizes[0]),
                          (f1_ref, fb1_ref, fsizes[1]),
                          (f2_ref, fb2_ref, fsizes[2])):
            Lk = P2 - k + 1
            m = None
            for j in range(k):
                z = jnp.dot(x3, fr[16 * j:16 * (j + 1), :],
                            preferred_element_type=jnp.float32)
                z = z.reshape(EB, P2, OUT)[:, j:j + Lk, :]
                m = z if m is None else m + z
            outs.append(jnp.max(m + fb[...][None], axis=1))
        cat = jnp.concatenate(outs, axis=-1)
        h1 = jax.nn.relu(jnp.dot(cat, l1w_ref[...],
                                 preferred_element_type=jnp.float32)
                         + l1b_ref[...])
        o_ref[...] = (jnp.dot(h1, l2w_ref[...],
                              preferred_element_type=jnp.float32)
                      + l2b_ref[...])

    cons = lambda shape: pl.BlockSpec(shape, lambda i: tuple(0 for _ in shape))
    return pl.pallas_call(
        body,
        grid=(NSEQ // EB,),
        in_specs=[
            pl.BlockSpec((EB * T, 1), lambda i: (i, 0)),
            cons((Vp, 64)),
            cons((5 * 64, 32)), cons((1, 32)),
            cons((3 * 32, 16)), cons((1, 16)),
            cons((fsizes[0] * 16, 128)), cons((fsizes[1] * 16, 128)),
            cons((fsizes[2] * 16, 128)),
            cons((1, 128)), cons((1, 128)), cons((1, 128)),
            cons((3 * 128, 128)), cons((1, 128)),
            cons((128, 128)), cons((1, 128)),
        ],
        out_specs=pl.BlockSpec((EB, 128), lambda i: (i, 0)),
        out_shape=jax.ShapeDtypeStruct((NSEQ, 128), jnp.float32),
    )


_enc_drug = _make_encoder(100, 64, (3, 5, 7))
_enc_target = _make_encoder(128, 26, (3, 6, 9))


def _seq_filter(p, idx, enc, V):
    Vp = max(16, 1 << (V - 1).bit_length())
    emb = jnp.zeros((Vp, 64), jnp.float32).at[:V].set(p['emb'])
    w1 = jnp.concatenate([p['c1w'][:, :, j].T for j in range(5)], axis=0)
    w2 = jnp.concatenate([p['c2w'][:, :, j].T for j in range(3)], axis=0)
    fws = [jnp.concatenate([fw[:, :, j].T for j in range(fw.shape[2])], axis=0)
           for fw in p['fw']]
    return enc(idx.astype(jnp.int32).reshape(-1, 1), emb,
               w1, p['c1b'][None], w2, p['c2b'][None],
               fws[0], fws[1], fws[2],
               p['fb'][0][None], p['fb'][1][None], p['fb'][2][None],
               p['l1w'].T, p['l1b'][None], p['l2w'].T, p['l2b'][None])


def kernel(params, drug_features, target_features, edges, all_edges):
    df = _seq_filter(params['drug'], drug_features, _enc_drug, 64)
    tf = _seq_filter(params['target'], target_features, _enc_target, 26)
    x = jnp.concatenate([df, tf], axis=0)
    xp = jnp.pad(x, ((0, NPAD - N), (0, 0)))

    src = all_edges[0].astype(jnp.int32)
    dst = all_edges[1].astype(jnp.int32)

    for lp in params['layers']:
        Wh = lp['gat_w'].T
        v1 = Wh @ lp['att_src']
        v2 = Wh @ lp['att_dst']
        Vp = jnp.zeros((D, D), jnp.float32).at[:, 0].set(v1).at[:, 1].set(v2)
        h, A = _pre_tc(xp, Wh, Vp)
        asv = A[:, 0]
        adv = A[:, 1]
        ee = jnp.exp(jax.nn.leaky_relu(asv[src] + adv[dst], negative_slope=0.2))
        accg = jax.ops.segment_sum(ee[:, None] * h[src], dst, num_segments=NPAD)
        den16 = jnp.pad(jax.ops.segment_sum(ee, dst, num_segments=NPAD)[:, None], ((0, 0), (0, 15)))
        accx = jax.ops.segment_sum(xp[src], dst, num_segments=NPAD)
        cnt16 = jnp.pad(jax.ops.segment_sum(jnp.ones_like(ee), dst, num_segments=NPAD)[:, None], ((0, 0), (0, 15)))
        xp = _post_tc(xp, h, accg, accx, A, den16, cnt16,
                      lp['sage_lw'].T, lp['sage_rw'].T,
                      lp['lin_w'][:, :D].T, lp['lin_w'][:, D:2 * D].T,
                      lp['lin_w'][:, 2 * D:].T,
                      lp['gat_b'][None, :], lp['sage_lb'][None, :],
                      lp['lin_b'][None, :])

    e0 = edges[0].astype(jnp.int32)
    e1 = edges[1].astype(jnp.int32)
    e0p = jnp.concatenate([e0, jnp.zeros((E_DEC_PAD - E_DEC,), jnp.int32)])
    e1p = jnp.concatenate([e1, jnp.zeros((E_DEC_PAD - E_DEC,), jnp.int32)])
    probs = _decode_sc(xp, e0p, e1p)
    return probs[:E_DEC]
